# baseline (speedup 1.0000x reference)
import functools

import jax
import jax.numpy as jnp
from jax import lax
from jax.experimental import pallas as pl
from jax.experimental.pallas import tpu as pltpu
from jax.experimental.pallas import tpu_sc as plsc

N = 10000
D = 128
H = 64
R = 3
E = 320000
ROWS = R * N
ACC_ROWS = N + 8
EW = E // 32
NC = 2
NS = 16
NW = NC * NS
CHUNK = 128
CHUNKS = 80
E_PS = CHUNKS * CHUNK
E_PAD = NW * E_PS
ROWS2D = E_PAD // CHUNK
RPW = 632
RPW_LAST = N - 15 * RPW
BN = 400
BR = 512
NBUF = 8


def _relu(x):
    return jnp.maximum(x, 0.0)


def _tables_body(x_ref, w1_ref, b1_ref, w2_ref, b2_ref, w3_ref, b3_ref,
                 wp_ref, t_ref):
    x = x_ref[...]
    for i in range(R):
        h = _relu(jnp.dot(x, w1_ref[i], preferred_element_type=jnp.float32)
                  + b1_ref[i])
        h = _relu(jnp.dot(h, w2_ref[i], preferred_element_type=jnp.float32)
                  + b2_ref[i])
        y = _relu(jnp.dot(h, w3_ref[i], preferred_element_type=jnp.float32)
                  + b3_ref[i])
        t_ref[i] = jnp.dot(y, wp_ref[i], preferred_element_type=jnp.float32)


def _run_tables(x, rel_params, w1r):
    w1s = jnp.stack([p['W1'] for p in rel_params])
    b1s = jnp.stack([p['b1'] for p in rel_params])[:, None]
    w2s = jnp.stack([p['W2'] for p in rel_params])
    b2s = jnp.stack([p['b2'] for p in rel_params])[:, None]
    w3s = jnp.stack([p['W3'] for p in rel_params])
    b3s = jnp.stack([p['b3'] for p in rel_params])[:, None]
    grid = (N // BN,)
    return pl.pallas_call(
        _tables_body,
        grid=grid,
        in_specs=[
            pl.BlockSpec((BN, D), lambda n: (n, 0)),
            pl.BlockSpec((R, D, H), lambda n: (0, 0, 0)),
            pl.BlockSpec((R, 1, H), lambda n: (0, 0, 0)),
            pl.BlockSpec((R, H, H), lambda n: (0, 0, 0)),
            pl.BlockSpec((R, 1, H), lambda n: (0, 0, 0)),
            pl.BlockSpec((R, H, D), lambda n: (0, 0, 0)),
            pl.BlockSpec((R, 1, D), lambda n: (0, 0, 0)),
            pl.BlockSpec((R, D, H), lambda n: (0, 0, 0)),
        ],
        out_specs=pl.BlockSpec((R, BN, H), lambda n: (0, n, 0)),
        out_shape=jax.ShapeDtypeStruct((R, N, H), jnp.float32),
    )(x, w1s, b1s, w2s, b2s, w3s, b3s, w1r)


def _index_body(src_ref, dst_ref, et_ref, ii_ref, oi_ref):
    pid = pl.program_id(0)
    src = src_ref[...]
    dst = dst_ref[...]
    et = et_ref[...]
    rows = lax.broadcasted_iota(jnp.int32, src.shape, 0) + pid * BR
    cols = lax.broadcasted_iota(jnp.int32, src.shape, 1)
    valid = (rows % CHUNKS) * CHUNK + cols < EW
    ii_ref[...] = jnp.where(valid, et * N + src, ROWS)
    oi_ref[...] = jnp.where(valid, dst, (rows * CHUNK + cols) % N)


def _run_indices(src_p, dst_p, et_p):
    grid = (ROWS2D // BR,)
    return pl.pallas_call(
        _index_body,
        grid=grid,
        in_specs=[
            pl.BlockSpec((BR, CHUNK), lambda b: (b, 0)),
            pl.BlockSpec((BR, CHUNK), lambda b: (b, 0)),
            pl.BlockSpec((BR, CHUNK), lambda b: (b, 0)),
        ],
        out_specs=[
            pl.BlockSpec((BR, CHUNK), lambda b: (b, 0)),
            pl.BlockSpec((BR, CHUNK), lambda b: (b, 0)),
        ],
        out_shape=[
            jax.ShapeDtypeStruct((ROWS2D, CHUNK), jnp.int32),
            jax.ShapeDtypeStruct((ROWS2D, CHUNK), jnp.int32),
        ],
    )(src_p, dst_p, et_p)


@functools.cache
def _build_sc_segment_sum():
    mesh = plsc.VectorSubcoreMesh(
        core_axis_name="c", subcore_axis_name="s",
        num_cores=NC, num_subcores=NS)

    @functools.partial(
        pl.kernel,
        out_type=jax.ShapeDtypeStruct((NC, N, H), jnp.float32),
        mesh=mesh,
        scratch_types=[
            pltpu.VMEM((CHUNKS, CHUNK), jnp.int32),
            pltpu.VMEM((CHUNKS, CHUNK), jnp.int32),
            pltpu.VMEM((NBUF, CHUNK, H), jnp.float32),
            pltpu.VMEM_SHARED((ACC_ROWS, H), jnp.float32),
            [pltpu.SemaphoreType.DMA] * NBUF,
        ],
        compiler_params=pltpu.CompilerParams(use_tc_tiling_on_sc=False),
    )
    def _sc_segment_sum(table_ref, ii_ref, oi_ref, z_ref, out_ref,
                        iidx, oidx, rows, acc, gsems):
        c = lax.axis_index("c")
        s = lax.axis_index("s")

        @pl.when(s < NS - 1)
        def _():
            pltpu.sync_copy(z_ref.at[pl.ds(0, RPW)],
                            acc.at[pl.ds(s * RPW, RPW)])

        @pl.when(s == NS - 1)
        def _():
            pltpu.sync_copy(z_ref.at[pl.ds(0, RPW_LAST + 8)],
                            acc.at[pl.ds(s * RPW, RPW_LAST + 8)])

        pltpu.sync_copy(ii_ref.at[c, s], iidx)
        pltpu.sync_copy(oi_ref.at[c, s], oidx)
        plsc.subcore_barrier()

        for b in range(NBUF):
            pltpu.async_copy(table_ref.at[iidx.at[b]], rows.at[b], gsems[b])

        def group(g, carry):
            for b in range(NBUF):
                j = g * NBUF + b
                pltpu.make_async_copy(table_ref.at[iidx.at[j]],
                                      rows.at[b], gsems[b]).wait()
                pltpu.sync_copy(rows.at[b], acc.at[oidx.at[j]], add=True)
                jn = j + NBUF

                @pl.when(jn < CHUNKS)
                def _():
                    pltpu.async_copy(table_ref.at[iidx.at[jn]],
                                     rows.at[b], gsems[b])
            return carry

        lax.fori_loop(0, CHUNKS // NBUF, group, 0)
        plsc.subcore_barrier()

        @pl.when(s < NS - 1)
        def _():
            pltpu.sync_copy(acc.at[pl.ds(s * RPW, RPW)],
                            out_ref.at[c, pl.ds(s * RPW, RPW)])

        @pl.when(s == NS - 1)
        def _():
            pltpu.sync_copy(acc.at[pl.ds(s * RPW, RPW_LAST)],
                            out_ref.at[c, pl.ds(s * RPW, RPW_LAST)])

    return _sc_segment_sum


def _update_body(x_ref, agg_ref, w1x_ref, b1_ref, w2_ref, b2_ref,
                 w3_ref, b3_ref, nt_ref, o_ref):
    x = x_ref[...]
    acc = (jnp.dot(_relu(x), w1x_ref[...], preferred_element_type=jnp.float32)
           + b1_ref[...])
    h1 = _relu(acc + agg_ref[0] + agg_ref[1])
    h2 = _relu(jnp.dot(h1, w2_ref[...], preferred_element_type=jnp.float32)
               + b2_ref[...])
    o = (jnp.dot(h2, w3_ref[...], preferred_element_type=jnp.float32)
         + b3_ref[...])
    nt = nt_ref[...]
    mask = jnp.logical_or(nt == 0.0, nt == 1.0).astype(jnp.float32)
    o_ref[...] = o * mask


def _run_update(x, agg, node_params, ntf):
    w1 = node_params['W1']
    w1x = w1[:D]
    b1 = node_params['b1'][None]
    w2 = node_params['W2']
    b2 = node_params['b2'][None]
    w3 = node_params['W3']
    b3 = node_params['b3'][None]
    grid = (N // BN,)
    return pl.pallas_call(
        _update_body,
        grid=grid,
        in_specs=[
            pl.BlockSpec((BN, D), lambda n: (n, 0)),
            pl.BlockSpec((NC, BN, H), lambda n: (0, n, 0)),
            pl.BlockSpec((D, H), lambda n: (0, 0)),
            pl.BlockSpec((1, H), lambda n: (0, 0)),
            pl.BlockSpec((H, H), lambda n: (0, 0)),
            pl.BlockSpec((1, H), lambda n: (0, 0)),
            pl.BlockSpec((H, D), lambda n: (0, 0)),
            pl.BlockSpec((1, D), lambda n: (0, 0)),
            pl.BlockSpec((BN, 1), lambda n: (n, 0)),
        ],
        out_specs=pl.BlockSpec((BN, D), lambda n: (n, 0)),
        out_shape=jax.ShapeDtypeStruct((N, D), jnp.float32),
    )(x, agg, w1x, b1, w2, b2, w3, b3, ntf)


def kernel(node_feature, params, edge_index, edge_type, node_type):
    src = edge_index[0]
    dst = edge_index[1]

    def per_worker(a):
        return jnp.pad(a.reshape(NW, EW),
                       ((0, 0), (0, E_PS - EW))).reshape(ROWS2D, CHUNK)

    src_p = per_worker(src)
    dst_p = per_worker(dst)
    et_p = per_worker(edge_type)

    w1 = params['node']['W1']
    w1r = w1[D:].reshape(R, D, H)
    tables = _run_tables(node_feature, params['rel'], w1r)
    ii, oi = _run_indices(src_p, dst_p, et_p)

    table_sc = jnp.concatenate(
        [tables.reshape(ROWS, H), jnp.zeros((8, H), jnp.float32)])
    agg = _build_sc_segment_sum()(
        table_sc,
        ii.reshape(NC, NS, CHUNKS, CHUNK),
        oi.reshape(NC, NS, CHUNKS, CHUNK),
        jnp.zeros((RPW + 8, H), jnp.float32),
    )

    ntf = node_type.astype(jnp.float32).reshape(N, 1)
    return _run_update(node_feature, agg, params['node'], ntf)


# speedup vs baseline: 1.8223x; 1.8223x over previous
import functools

import jax
import jax.numpy as jnp
from jax import lax
from jax.experimental import pallas as pl
from jax.experimental.pallas import tpu as pltpu
from jax.experimental.pallas import tpu_sc as plsc

N = 10000
D = 128
H = 64
R = 3
E = 320000
ROWS = R * N
SPARE = 232
ACC_ROWS = N + SPARE
EW = E // 32
NC = 2
NS = 16
NW = NC * NS
CHUNK = 128
CHUNKS = 80
E_PS = CHUNKS * CHUNK
E_PAD = NW * E_PS
ROWS2D = E_PAD // CHUNK
RPW = 632
RPW_LAST = N - 15 * RPW
BN = 400
BR = 512
NBUF = 8


def _relu(x):
    return jnp.maximum(x, 0.0)


def _tables_body(x_ref, w1_ref, b1_ref, w2_ref, b2_ref, w3_ref, b3_ref,
                 wp_ref, t_ref):
    x = x_ref[...]
    for i in range(R):
        h = _relu(jnp.dot(x, w1_ref[i], preferred_element_type=jnp.float32)
                  + b1_ref[i])
        h = _relu(jnp.dot(h, w2_ref[i], preferred_element_type=jnp.float32)
                  + b2_ref[i])
        y = _relu(jnp.dot(h, w3_ref[i], preferred_element_type=jnp.float32)
                  + b3_ref[i])
        t_ref[i] = jnp.dot(y, wp_ref[i], preferred_element_type=jnp.float32)


def _run_tables(x, rel_params, w1r):
    w1s = jnp.stack([p['W1'] for p in rel_params])
    b1s = jnp.stack([p['b1'] for p in rel_params])[:, None]
    w2s = jnp.stack([p['W2'] for p in rel_params])
    b2s = jnp.stack([p['b2'] for p in rel_params])[:, None]
    w3s = jnp.stack([p['W3'] for p in rel_params])
    b3s = jnp.stack([p['b3'] for p in rel_params])[:, None]
    grid = (N // BN,)
    return pl.pallas_call(
        _tables_body,
        grid=grid,
        in_specs=[
            pl.BlockSpec((BN, D), lambda n: (n, 0)),
            pl.BlockSpec((R, D, H), lambda n: (0, 0, 0)),
            pl.BlockSpec((R, 1, H), lambda n: (0, 0, 0)),
            pl.BlockSpec((R, H, H), lambda n: (0, 0, 0)),
            pl.BlockSpec((R, 1, H), lambda n: (0, 0, 0)),
            pl.BlockSpec((R, H, D), lambda n: (0, 0, 0)),
            pl.BlockSpec((R, 1, D), lambda n: (0, 0, 0)),
            pl.BlockSpec((R, D, H), lambda n: (0, 0, 0)),
        ],
        out_specs=pl.BlockSpec((R, BN, H), lambda n: (0, n, 0)),
        out_shape=jax.ShapeDtypeStruct((R, N, H), jnp.float32),
    )(x, w1s, b1s, w2s, b2s, w3s, b3s, w1r)


def _index_body(src_ref, dst_ref, et_ref, ii_ref, oi_ref):
    pid = pl.program_id(0)
    src = src_ref[...]
    dst = dst_ref[...]
    et = et_ref[...]
    rows = lax.broadcasted_iota(jnp.int32, src.shape, 0) + pid * BR
    cols = lax.broadcasted_iota(jnp.int32, src.shape, 1)
    lin = rows * CHUNK + cols
    valid = (rows % CHUNKS) * CHUNK + cols < EW
    ii_ref[...] = jnp.where(valid, et * N + src, lin % ROWS)
    oi_ref[...] = jnp.where(valid, dst, N + lin % SPARE)


def _run_indices(src_p, dst_p, et_p):
    grid = (ROWS2D // BR,)
    return pl.pallas_call(
        _index_body,
        grid=grid,
        in_specs=[
            pl.BlockSpec((BR, CHUNK), lambda b: (b, 0)),
            pl.BlockSpec((BR, CHUNK), lambda b: (b, 0)),
            pl.BlockSpec((BR, CHUNK), lambda b: (b, 0)),
        ],
        out_specs=[
            pl.BlockSpec((BR, CHUNK), lambda b: (b, 0)),
            pl.BlockSpec((BR, CHUNK), lambda b: (b, 0)),
        ],
        out_shape=[
            jax.ShapeDtypeStruct((ROWS2D, CHUNK), jnp.int32),
            jax.ShapeDtypeStruct((ROWS2D, CHUNK), jnp.int32),
        ],
    )(src_p, dst_p, et_p)


@functools.cache
def _build_sc_segment_sum():
    mesh = plsc.VectorSubcoreMesh(
        core_axis_name="c", subcore_axis_name="s",
        num_cores=NC, num_subcores=NS)

    @functools.partial(
        pl.kernel,
        out_type=jax.ShapeDtypeStruct((NC, N, H), jnp.float32),
        mesh=mesh,
        scratch_types=[
            pltpu.VMEM((CHUNKS, CHUNK), jnp.int32),
            pltpu.VMEM((CHUNKS, CHUNK), jnp.int32),
            pltpu.VMEM((NBUF, CHUNK, H), jnp.float32),
            pltpu.VMEM_SHARED((ACC_ROWS, H), jnp.float32),
            [pltpu.SemaphoreType.DMA] * NBUF,
        ],
        compiler_params=pltpu.CompilerParams(use_tc_tiling_on_sc=False),
    )
    def _sc_segment_sum(table_ref, ii_ref, oi_ref, z_ref, out_ref,
                        iidx, oidx, rows, acc, gsems):
        c = lax.axis_index("c")
        s = lax.axis_index("s")

        @pl.when(s < NS - 1)
        def _():
            pltpu.sync_copy(z_ref.at[pl.ds(0, RPW)],
                            acc.at[pl.ds(s * RPW, RPW)])

        @pl.when(s == NS - 1)
        def _():
            pltpu.sync_copy(z_ref.at[pl.ds(0, RPW_LAST + 8)],
                            acc.at[pl.ds(s * RPW, RPW_LAST + 8)])

        pltpu.sync_copy(ii_ref.at[c, s], iidx)
        pltpu.sync_copy(oi_ref.at[c, s], oidx)
        plsc.subcore_barrier()

        for b in range(NBUF):
            pltpu.async_copy(table_ref.at[iidx.at[b]], rows.at[b], gsems[b])

        def group(g, carry):
            for b in range(NBUF):
                j = g * NBUF + b
                pltpu.make_async_copy(table_ref.at[iidx.at[j]],
                                      rows.at[b], gsems[b]).wait()
                pltpu.sync_copy(rows.at[b], acc.at[oidx.at[j]], add=True)
                jn = j + NBUF

                @pl.when(jn < CHUNKS)
                def _():
                    pltpu.async_copy(table_ref.at[iidx.at[jn]],
                                     rows.at[b], gsems[b])
            return carry

        lax.fori_loop(0, CHUNKS // NBUF, group, 0)
        plsc.subcore_barrier()

        @pl.when(s < NS - 1)
        def _():
            pltpu.sync_copy(acc.at[pl.ds(s * RPW, RPW)],
                            out_ref.at[c, pl.ds(s * RPW, RPW)])

        @pl.when(s == NS - 1)
        def _():
            pltpu.sync_copy(acc.at[pl.ds(s * RPW, RPW_LAST)],
                            out_ref.at[c, pl.ds(s * RPW, RPW_LAST)])

    return _sc_segment_sum


def _update_body(x_ref, agg_ref, w1x_ref, b1_ref, w2_ref, b2_ref,
                 w3_ref, b3_ref, nt_ref, o_ref):
    x = x_ref[...]
    acc = (jnp.dot(_relu(x), w1x_ref[...], preferred_element_type=jnp.float32)
           + b1_ref[...])
    h1 = _relu(acc + agg_ref[0] + agg_ref[1])
    h2 = _relu(jnp.dot(h1, w2_ref[...], preferred_element_type=jnp.float32)
               + b2_ref[...])
    o = (jnp.dot(h2, w3_ref[...], preferred_element_type=jnp.float32)
         + b3_ref[...])
    nt = nt_ref[...]
    mask = jnp.logical_or(nt == 0.0, nt == 1.0).astype(jnp.float32)
    o_ref[...] = o * mask


def _run_update(x, agg, node_params, ntf):
    w1 = node_params['W1']
    w1x = w1[:D]
    b1 = node_params['b1'][None]
    w2 = node_params['W2']
    b2 = node_params['b2'][None]
    w3 = node_params['W3']
    b3 = node_params['b3'][None]
    grid = (N // BN,)
    return pl.pallas_call(
        _update_body,
        grid=grid,
        in_specs=[
            pl.BlockSpec((BN, D), lambda n: (n, 0)),
            pl.BlockSpec((NC, BN, H), lambda n: (0, n, 0)),
            pl.BlockSpec((D, H), lambda n: (0, 0)),
            pl.BlockSpec((1, H), lambda n: (0, 0)),
            pl.BlockSpec((H, H), lambda n: (0, 0)),
            pl.BlockSpec((1, H), lambda n: (0, 0)),
            pl.BlockSpec((H, D), lambda n: (0, 0)),
            pl.BlockSpec((1, D), lambda n: (0, 0)),
            pl.BlockSpec((BN, 1), lambda n: (n, 0)),
        ],
        out_specs=pl.BlockSpec((BN, D), lambda n: (n, 0)),
        out_shape=jax.ShapeDtypeStruct((N, D), jnp.float32),
    )(x, agg, w1x, b1, w2, b2, w3, b3, ntf)


def kernel(node_feature, params, edge_index, edge_type, node_type):
    src = edge_index[0]
    dst = edge_index[1]

    def per_worker(a):
        return jnp.pad(a.reshape(NW, EW),
                       ((0, 0), (0, E_PS - EW))).reshape(ROWS2D, CHUNK)

    src_p = per_worker(src)
    dst_p = per_worker(dst)
    et_p = per_worker(edge_type)

    w1 = params['node']['W1']
    w1r = w1[D:].reshape(R, D, H)
    tables = _run_tables(node_feature, params['rel'], w1r)
    ii, oi = _run_indices(src_p, dst_p, et_p)

    agg = _build_sc_segment_sum()(
        tables.reshape(ROWS, H),
        ii.reshape(NC, NS, CHUNKS, CHUNK),
        oi.reshape(NC, NS, CHUNKS, CHUNK),
        jnp.zeros((RPW + 8, H), jnp.float32),
    )

    ntf = node_type.astype(jnp.float32).reshape(N, 1)
    return _run_update(node_feature, agg, params['node'], ntf)
